# Initial kernel scaffold; baseline (speedup 1.0000x reference)
import jax
import jax.numpy as jnp
from jax.experimental import pallas as pl


def kernel(x, b1, b2, b3, b3_, b4, b_f, b_max, w1, w2, w3, w3_, w4, w_f, w_max):
    raise NotImplementedError("write your pallas kernel here")


# speedup vs baseline: 2.4268x; 2.4268x over previous
import functools

import numpy as np
import jax
import jax.numpy as jnp
from jax import lax
from jax.experimental import pallas as pl
from jax.experimental.pallas import tpu as pltpu


def _bilinear_weights(out_size, in_size):
    a = np.zeros((out_size, in_size), dtype=np.float32)
    scale = in_size / out_size
    for i in range(out_size):
        src = max((i + 0.5) * scale - 0.5, 0.0)
        lo = min(int(src), in_size - 1)
        hi = min(lo + 1, in_size - 1)
        frac = src - lo
        a[i, lo] += 1.0 - frac
        a[i, hi] += frac
    return jnp.asarray(a)


def _esa_kernel(x_ref, w1_ref, b1_ref, w2_ref, b2_ref, wm_ref, bm_ref,
                w3_ref, b3_ref, w3p_ref, b3p_ref, wf_ref, bf_ref,
                w4_ref, b4_ref, ah_ref, awt_ref, o_ref, *, f, H, W, H2, Hm):
    xb = x_ref[0]

    c1f = jnp.dot(w1_ref[...], xb,
                  preferred_element_type=jnp.float32) + b1_ref[...]
    c13 = c1f.reshape(f, H, W)

    ph = c13.reshape(f, H // 2, 2, W)
    rows = [ph[:, 0:H2, 0, :], ph[:, 0:H2, 1, :], ph[:, 1:H2 + 1, 0, :]]
    jj = lax.broadcasted_iota(jnp.int32, (W, H2), 0)
    xx = lax.broadcasted_iota(jnp.int32, (W, H2), 1)
    taps = []
    for ky in range(3):
        for kx in range(3):
            ck = jnp.where(jj == 2 * xx + kx, 1.0, 0.0).astype(jnp.float32)
            taps.append(lax.dot_general(
                rows[ky], ck, (((2,), (0,)), ((), ())),
                precision=lax.Precision.HIGHEST,
                preferred_element_type=jnp.float32))
    patches = jnp.concatenate(taps, axis=0).reshape(9 * f, H2 * H2)
    c1 = (jnp.dot(w2_ref[...], patches,
                  preferred_element_type=jnp.float32)
          + b2_ref[...]).reshape(f, H2, H2)

    cols = [jnp.max(c1[:, :, 3 * i:3 * i + 7], axis=2, keepdims=True)
            for i in range(Hm)]
    cm = jnp.concatenate(cols, axis=2)
    rws = [jnp.max(cm[:, 3 * i:3 * i + 7, :], axis=1, keepdims=True)
           for i in range(Hm)]
    vm = jnp.concatenate(rws, axis=1)

    def conv3x3_same(v, w_ref, b_ref, relu):
        vp = jnp.pad(v, ((0, 0), (1, 1), (1, 1)))
        tp = jnp.concatenate([vp[:, ky:ky + Hm, kx:kx + Hm]
                              for ky in range(3) for kx in range(3)],
                             axis=0).reshape(9 * f, Hm * Hm)
        y = (jnp.dot(w_ref[...], tp,
                     preferred_element_type=jnp.float32)
             + b_ref[...]).reshape(f, Hm, Hm)
        return jnp.maximum(y, 0.0) if relu else y

    vr = conv3x3_same(vm, wm_ref, bm_ref, True)
    c3 = conv3x3_same(vr, w3_ref, b3_ref, True)
    c3 = conv3x3_same(c3, w3p_ref, b3p_ref, False)

    ahb = jnp.broadcast_to(ah_ref[...][None], (f, H, Hm))
    t = lax.dot_general(ahb, c3, (((2,), (1,)), ((0,), (0,))),
                        preferred_element_type=jnp.float32)
    c3u = lax.dot_general(t, awt_ref[...], (((2,), (0,)), ((), ())),
                          preferred_element_type=jnp.float32)

    cf = jnp.dot(wf_ref[...], c1f,
                 preferred_element_type=jnp.float32) + bf_ref[...]
    s = c3u + cf.reshape(f, H, W)
    c4 = lax.dot_general(w4_ref[...], s, (((1,), (0,)), ((), ())),
                         preferred_element_type=jnp.float32) + b4_ref[...]
    gate = 1.0 / (1.0 + jnp.exp(-c4))
    o_ref[0] = (xb.reshape(o_ref.shape[1:]) * gate).astype(o_ref.dtype)


def kernel(x, b1, b2, b3, b3_, b4, b_f, b_max, w1, w2, w3, w3_, w4, w_f, w_max):
    N, C, H, W = x.shape
    f = C // 4
    H2 = (H - 3) // 2 + 1
    Hm = (H2 - 7) // 3 + 1
    S = H * W

    def tap_layout(w):
        return jnp.transpose(w, (0, 2, 3, 1)).reshape(w.shape[0], 9 * w.shape[1])

    ah = _bilinear_weights(H, Hm)
    awt = _bilinear_weights(W, Hm).T

    col = lambda b: b.reshape(b.shape[0], 1)
    full = lambda shape: pl.BlockSpec(shape, lambda n: tuple(0 for _ in shape))

    return pl.pallas_call(
        functools.partial(_esa_kernel, f=f, H=H, W=W, H2=H2, Hm=Hm),
        out_shape=jax.ShapeDtypeStruct((N, C, H, W), x.dtype),
        grid=(N,),
        in_specs=[
            pl.BlockSpec((1, C, S), lambda n: (n, 0, 0)),
            full((f, C)), full((f, 1)),
            full((f, 9 * f)), full((f, 1)),
            full((f, 9 * f)), full((f, 1)),
            full((f, 9 * f)), full((f, 1)),
            full((f, 9 * f)), full((f, 1)),
            full((f, f)), full((f, 1)),
            full((C, f)), full((C, 1, 1)),
            full((H, Hm)), full((Hm, W)),
        ],
        out_specs=pl.BlockSpec((1, C, H, W), lambda n: (n, 0, 0, 0)),
        compiler_params=pltpu.CompilerParams(
            dimension_semantics=("parallel",),
            vmem_limit_bytes=100 * 1024 * 1024),
    )(x.reshape(N, C, S),
      w1.reshape(f, C), col(b1),
      tap_layout(w2), col(b2),
      tap_layout(w_max), col(b_max),
      tap_layout(w3), col(b3),
      tap_layout(w3_), col(b3_),
      w_f.reshape(f, f), col(b_f),
      w4.reshape(C, f), b4.reshape(C, 1, 1),
      ah, awt)


# speedup vs baseline: 2.7015x; 1.1132x over previous
import functools

import numpy as np
import jax
import jax.numpy as jnp
from jax import lax
from jax.experimental import pallas as pl
from jax.experimental.pallas import tpu as pltpu


def _bilinear_weights(out_size, in_size):
    a = np.zeros((out_size, in_size), dtype=np.float32)
    scale = in_size / out_size
    for i in range(out_size):
        src = max((i + 0.5) * scale - 0.5, 0.0)
        lo = min(int(src), in_size - 1)
        hi = min(lo + 1, in_size - 1)
        frac = src - lo
        a[i, lo] += 1.0 - frac
        a[i, hi] += frac
    return a


def _upsample_matrix(H, W, Hm, Wm):
    ah = _bilinear_weights(H, Hm)
    aw = _bilinear_weights(W, Wm)
    u = np.einsum("hm,wk->mkhw", ah, aw).reshape(Hm * Wm, H * W)
    return jnp.asarray(u)


def _esa_kernel(x_ref, w1_ref, b1_ref, w2_ref, b2_ref, wm_ref, bm_ref,
                w3_ref, b3_ref, w3p_ref, b3p_ref, wf_ref, bf_ref,
                w4_ref, b4_ref, u_ref, o_ref, c13_ref, tap_ref, tap9_ref,
                *, f, H, W, H2, Hm):
    xb = x_ref[0]

    c1f = jnp.dot(w1_ref[...], xb,
                  preferred_element_type=jnp.float32) + b1_ref[...]

    c13_ref[...] = c1f.reshape(f, H, W)
    jj = lax.broadcasted_iota(jnp.int32, (W, 3 * H2), 0)
    xx = lax.broadcasted_iota(jnp.int32, (W, 3 * H2), 1)
    ck = jnp.where(jj == 2 * (xx % H2) + xx // H2, 1.0, 0.0).astype(jnp.float32)
    for ky in range(3):
        rows = c13_ref[:, pl.ds(ky, H2, 2), :]
        z = lax.dot_general(rows, ck, (((2,), (0,)), ((), ())),
                            precision=lax.Precision.HIGHEST,
                            preferred_element_type=jnp.float32)
        for kx in range(3):
            tap_ref[(3 * ky + kx) * f:(3 * ky + kx + 1) * f, :] = (
                z[:, :, kx * H2:(kx + 1) * H2].reshape(f, H2 * H2))
    c1 = (jnp.dot(w2_ref[...], tap_ref[...],
                  preferred_element_type=jnp.float32)
          + b2_ref[...]).reshape(f, H2, H2)

    cols = [jnp.max(c1[:, :, 3 * i:3 * i + 7], axis=2, keepdims=True)
            for i in range(Hm)]
    cm = jnp.concatenate(cols, axis=2)
    rws = [jnp.max(cm[:, 3 * i:3 * i + 7, :], axis=1, keepdims=True)
           for i in range(Hm)]
    vm = jnp.concatenate(rws, axis=1)

    def conv3x3_same(v, w_ref, b_ref, relu):
        vp = jnp.pad(v, ((0, 0), (1, 1), (1, 1)))
        for t, (ky, kx) in enumerate((ky, kx) for ky in range(3)
                                     for kx in range(3)):
            tap9_ref[t * f:(t + 1) * f, :] = (
                vp[:, ky:ky + Hm, kx:kx + Hm].reshape(f, Hm * Hm))
        y = (jnp.dot(w_ref[...], tap9_ref[...],
                     preferred_element_type=jnp.float32)
             + b_ref[...]).reshape(f, Hm, Hm)
        return jnp.maximum(y, 0.0) if relu else y

    vr = conv3x3_same(vm, wm_ref, bm_ref, True)
    c3 = conv3x3_same(vr, w3_ref, b3_ref, True)
    c3 = conv3x3_same(c3, w3p_ref, b3p_ref, False)

    c3flat = c3.reshape(f, Hm * Hm)
    c3u = jnp.dot(c3flat, u_ref[...], preferred_element_type=jnp.float32)

    cf = jnp.dot(wf_ref[...], c1f,
                 preferred_element_type=jnp.float32) + bf_ref[...]
    c4 = jnp.dot(w4_ref[...], c3u + cf,
                 preferred_element_type=jnp.float32) + b4_ref[...]
    gate = 1.0 / (1.0 + jnp.exp(-c4))
    o_ref[0] = (xb * gate).astype(o_ref.dtype)


def kernel(x, b1, b2, b3, b3_, b4, b_f, b_max, w1, w2, w3, w3_, w4, w_f, w_max):
    N, C, H, W = x.shape
    f = C // 4
    H2 = (H - 3) // 2 + 1
    Hm = (H2 - 7) // 3 + 1
    S = H * W

    def tap_layout(w):
        return jnp.transpose(w, (0, 2, 3, 1)).reshape(w.shape[0], 9 * w.shape[1])

    u = _upsample_matrix(H, W, Hm, Hm)

    col = lambda b: b.reshape(b.shape[0], 1)
    full = lambda shape: pl.BlockSpec(shape, lambda n: tuple(0 for _ in shape))

    out = pl.pallas_call(
        functools.partial(_esa_kernel, f=f, H=H, W=W, H2=H2, Hm=Hm),
        out_shape=jax.ShapeDtypeStruct((N, C, S), x.dtype),
        grid=(N,),
        in_specs=[
            pl.BlockSpec((1, C, S), lambda n: (n, 0, 0)),
            full((f, C)), full((f, 1)),
            full((f, 9 * f)), full((f, 1)),
            full((f, 9 * f)), full((f, 1)),
            full((f, 9 * f)), full((f, 1)),
            full((f, 9 * f)), full((f, 1)),
            full((f, f)), full((f, 1)),
            full((C, f)), full((C, 1)),
            full((Hm * Hm, S)),
        ],
        out_specs=pl.BlockSpec((1, C, S), lambda n: (n, 0, 0)),
        scratch_shapes=[pltpu.VMEM((f, H, W), jnp.float32),
                        pltpu.VMEM((9 * f, H2 * H2), jnp.float32),
                        pltpu.VMEM((9 * f, Hm * Hm), jnp.float32)],
        compiler_params=pltpu.CompilerParams(
            dimension_semantics=("parallel",),
            vmem_limit_bytes=100 * 1024 * 1024),
    )(x.reshape(N, C, S),
      w1.reshape(f, C), col(b1),
      tap_layout(w2), col(b2),
      tap_layout(w_max), col(b_max),
      tap_layout(w3), col(b3),
      tap_layout(w3_), col(b3_),
      w_f.reshape(f, f), col(b_f),
      w4.reshape(C, f), col(b4),
      u)
    return out.reshape(N, C, H, W)
